# baseline (speedup 1.0000x reference)
import functools

import jax
import jax.numpy as jnp
from jax import lax
from jax.experimental import pallas as pl
from jax.experimental.pallas import tpu as pltpu, tpu_sc as plsc

B, S, H = 4, 4096, 2048
N = B * S
K = int(0.8 * N)
BS = 1024
NB = N // BS
ROWS, COLS = 128, 128
RPB = BS // COLS
BETA_DENOM = float(B * (S - 1))
AUX_W = 0.01


def _router_body(x_ref, w_ref, scores_ref, mask_ref, aux_ref):
    i = pl.program_id(0)
    x = x_ref[...].astype(jnp.bfloat16).astype(jnp.float32)
    w = w_ref[...].astype(jnp.bfloat16).astype(jnp.float32)
    raw = jax.lax.dot_general(
        x, w, (((1,), (1,)), ((), ())),
        preferred_element_type=jnp.float32)[:, 0]
    s = jax.nn.sigmoid(raw)
    scores_ref[pl.ds(i * RPB, RPB), :] = s.reshape(RPB, COLS)

    @pl.when(i == NB - 1)
    def _():
        sc = scores_ref[...]
        bits = jax.lax.bitcast_convert_type(sc, jnp.int32)

        def srch(j, ans):
            cand = ans | jnp.left_shift(jnp.int32(1), 30 - j)
            cnt = jnp.sum((bits < cand).astype(jnp.int32))
            return jnp.where(cnt >= K, ans, cand)

        tbits = jax.lax.fori_loop(0, 31, srch, jnp.int32(0))
        m = (bits >= tbits).astype(jnp.float32)
        mask_ref[...] = m

        rate = jnp.sum(m) * (1.0 / N)
        d_in = jnp.sum(jnp.abs(m[:, 1:] - m[:, :-1]))
        left = m[1:, 0:1]
        right = m[:-1, COLS - 1:COLS]
        r_idx = jax.lax.broadcasted_iota(jnp.int32, (ROWS - 1, 1), 0)
        valid = (r_idx % 32 != 31).astype(jnp.float32)
        d_b = jnp.sum(jnp.abs(left - right) * valid)
        cons = (d_in + d_b) * (1.0 / BETA_DENOM)
        sparsity = jnp.maximum(0.1 - rate, 0.0)
        over = jnp.maximum(rate - 0.9, 0.0)
        aux_ref[...] = ((cons + sparsity + over) * AUX_W).reshape(1, 1)


def _router(x, w, interpret=False):
    return pl.pallas_call(
        _router_body,
        grid=(NB,),
        in_specs=[
            pl.BlockSpec((BS, H), lambda i: (i, 0)),
            pl.BlockSpec((1, H), lambda i: (0, 0)),
        ],
        out_specs=[
            pl.BlockSpec((ROWS, COLS), lambda i: (0, 0)),
            pl.BlockSpec((ROWS, COLS), lambda i: (0, 0)),
            pl.BlockSpec((1, 1), lambda i: (0, 0)),
        ],
        out_shape=[
            jax.ShapeDtypeStruct((ROWS, COLS), jnp.float32),
            jax.ShapeDtypeStruct((ROWS, COLS), jnp.float32),
            jax.ShapeDtypeStruct((1, 1), jnp.float32),
        ],
        interpret=interpret,
    )(x, w)


SC_R = 8192
SC_RW = SC_R // 32
SC_CH = 16
SC_NCH = SC_RW // SC_CH
SC_NBUF = 3

_sc_mesh = plsc.VectorSubcoreMesh(core_axis_name="c", subcore_axis_name="s")


@functools.partial(
    pl.kernel, mesh=_sc_mesh,
    out_type=jax.ShapeDtypeStruct((32, 16), jnp.float32),
    scratch_types=[
        pltpu.VMEM((SC_NBUF, SC_CH, 2048), jnp.float32),
        pltpu.VMEM((16,), jnp.float32),
        pltpu.SemaphoreType.DMA,
        pltpu.SemaphoreType.DMA,
        pltpu.SemaphoreType.DMA,
    ],
)
def _sc_stream(x_hbm, out_hbm, xb, vv, s0, s1, s2):
    wid = lax.axis_index("s") * 2 + lax.axis_index("c")
    base = wid * SC_RW
    sems = [s0, s1, s2]
    cps = []
    for c in range(SC_NCH):
        if c >= SC_NBUF:
            cps[c - SC_NBUF].wait()
        cps.append(pltpu.async_copy(
            x_hbm.at[pl.ds(base + c * SC_CH, SC_CH), :],
            xb.at[c % SC_NBUF], sems[c % SC_NBUF]))
    for c in range(SC_NCH - SC_NBUF, SC_NCH):
        cps[c].wait()
    vv[...] = xb[0, 0, 0:16]
    pltpu.sync_copy(vv, out_hbm.at[wid])


def kernel(hidden_states, recursion_step, W):
    x = hidden_states.reshape(N, H)
    sc_out = _sc_stream(x[:SC_R])
    scores, mask, aux = _router(x, W)
    a = jnp.where(sc_out[0, 0] * 0.0 == 0.0, aux[0, 0], sc_out[0, 0])
    return (scores.reshape(B, S), mask.reshape(B, S), a)


# speedup vs baseline: 2.5853x; 2.5853x over previous
import jax
import jax.numpy as jnp
from jax.experimental import pallas as pl

B, S, H = 4, 4096, 2048
N = B * S
K = int(0.8 * N)
BS = 1024
NB = N // BS // 2
ROWS, COLS = 128, 128
RPB = BS // COLS
BETA_DENOM = float(B * (S - 1))
AUX_W = 0.01


def _router_body(x1_ref, x2_ref, w_ref, scores_ref, mask_ref, aux_ref):
    i = pl.program_id(0)
    w = w_ref[...].astype(jnp.bfloat16).astype(jnp.float32)
    for x_ref, roff in ((x1_ref, 0), (x2_ref, ROWS // 2)):
        x = x_ref[...].astype(jnp.bfloat16).astype(jnp.float32)
        raw = jax.lax.dot_general(
            x, w, (((1,), (1,)), ((), ())),
            preferred_element_type=jnp.float32)[:, 0]
        s = jax.nn.sigmoid(raw)
        scores_ref[pl.ds(roff + i * RPB, RPB), :] = s.reshape(RPB, COLS)

    @pl.when(i == NB - 1)
    def _():
        sc = scores_ref[...]
        bits = jax.lax.bitcast_convert_type(sc, jnp.int32)

        def srch(j, ans):
            cand = ans | jnp.left_shift(jnp.int32(1), 30 - j)
            cnt = jnp.sum((bits < cand).astype(jnp.int32))
            return jnp.where(cnt >= K, ans, cand)

        tbits = jax.lax.fori_loop(0, 31, srch, jnp.int32(0))
        m = (bits >= tbits).astype(jnp.float32)
        mask_ref[...] = m

        rate = jnp.sum(m) * (1.0 / N)
        d_in = jnp.sum(jnp.abs(m[:, 1:] - m[:, :-1]))
        left = m[1:, 0:1]
        right = m[:-1, COLS - 1:COLS]
        r_idx = jax.lax.broadcasted_iota(jnp.int32, (ROWS - 1, 1), 0)
        valid = (r_idx % 32 != 31).astype(jnp.float32)
        d_b = jnp.sum(jnp.abs(left - right) * valid)
        cons = (d_in + d_b) * (1.0 / BETA_DENOM)
        sparsity = jnp.maximum(0.1 - rate, 0.0)
        over = jnp.maximum(rate - 0.9, 0.0)
        aux_ref[...] = ((cons + sparsity + over) * AUX_W).reshape(1, 1)


def _router(x, w, interpret=False):
    return pl.pallas_call(
        _router_body,
        grid=(NB,),
        in_specs=[
            pl.BlockSpec((BS, H), lambda i: (i, 0)),
            pl.BlockSpec((BS, H), lambda i: (i + NB, 0)),
            pl.BlockSpec((1, H), lambda i: (0, 0)),
        ],
        out_specs=[
            pl.BlockSpec((ROWS, COLS), lambda i: (0, 0)),
            pl.BlockSpec((ROWS, COLS), lambda i: (0, 0)),
            pl.BlockSpec((1, 1), lambda i: (0, 0)),
        ],
        out_shape=[
            jax.ShapeDtypeStruct((ROWS, COLS), jnp.float32),
            jax.ShapeDtypeStruct((ROWS, COLS), jnp.float32),
            jax.ShapeDtypeStruct((1, 1), jnp.float32),
        ],
        interpret=interpret,
    )(x, x, w)


def kernel(hidden_states, recursion_step, W):
    x = hidden_states.reshape(N, H)
    scores, mask, aux = _router(x, W)
    return (scores.reshape(B, S), mask.reshape(B, S), aux[0, 0])


# speedup vs baseline: 2.6811x; 1.0371x over previous
import jax
import jax.numpy as jnp
from jax.experimental import pallas as pl

B, S, H = 4, 4096, 2048
N = B * S
K = int(0.8 * N)
BS = 1024
NB = N // BS
ROWS, COLS = 128, 128
RPB = BS // COLS
BETA_DENOM = float(B * (S - 1))
AUX_W = 0.01


def _router_body(x_ref, w_ref, scores_ref, mask_ref, aux_ref):
    i = pl.program_id(0)
    x = x_ref[...].astype(jnp.bfloat16).astype(jnp.float32)
    w = w_ref[...].astype(jnp.bfloat16).astype(jnp.float32)
    raw = jax.lax.dot_general(
        x, w, (((1,), (1,)), ((), ())),
        preferred_element_type=jnp.float32)[:, 0]
    s = jax.nn.sigmoid(raw)
    scores_ref[pl.ds(i * RPB, RPB), :] = s.reshape(RPB, COLS)

    @pl.when(i == NB - 1)
    def _():
        sc = scores_ref[...]
        bits = jax.lax.bitcast_convert_type(sc, jnp.int32)

        def srch(j, ans):
            cand = ans | jnp.left_shift(jnp.int32(1), 30 - j)
            cnt = jnp.sum((bits < cand).astype(jnp.int32))
            return jnp.where(cnt >= K, ans, cand)

        tbits = jax.lax.fori_loop(0, 31, srch, jnp.int32(0))
        m = (bits >= tbits).astype(jnp.float32)
        mask_ref[...] = m

        rate = jnp.sum(m) * (1.0 / N)
        d_in = jnp.sum(jnp.abs(m[:, 1:] - m[:, :-1]))
        left = m[1:, 0:1]
        right = m[:-1, COLS - 1:COLS]
        r_idx = jax.lax.broadcasted_iota(jnp.int32, (ROWS - 1, 1), 0)
        valid = (r_idx % 32 != 31).astype(jnp.float32)
        d_b = jnp.sum(jnp.abs(left - right) * valid)
        cons = (d_in + d_b) * (1.0 / BETA_DENOM)
        sparsity = jnp.maximum(0.1 - rate, 0.0)
        over = jnp.maximum(rate - 0.9, 0.0)
        aux_ref[...] = ((cons + sparsity + over) * AUX_W).reshape(1, 1)


def _router(x, w, interpret=False):
    return pl.pallas_call(
        _router_body,
        grid=(NB,),
        in_specs=[
            pl.BlockSpec((BS, H), lambda i: (i, 0)),
            pl.BlockSpec((1, H), lambda i: (0, 0)),
        ],
        out_specs=[
            pl.BlockSpec((ROWS, COLS), lambda i: (0, 0)),
            pl.BlockSpec((ROWS, COLS), lambda i: (0, 0)),
            pl.BlockSpec((1, 1), lambda i: (0, 0)),
        ],
        out_shape=[
            jax.ShapeDtypeStruct((ROWS, COLS), jnp.float32),
            jax.ShapeDtypeStruct((ROWS, COLS), jnp.float32),
            jax.ShapeDtypeStruct((1, 1), jnp.float32),
        ],
        interpret=interpret,
    )(x, w)


def kernel(hidden_states, recursion_step, W):
    x = hidden_states.reshape(N, H)
    scores, mask, aux = _router(x, W)
    return (scores.reshape(B, S), mask.reshape(B, S), aux[0, 0])
